# baseline (device time: 1529677 ns/iter reference)
import jax
import jax.numpy as jnp
from jax import lax
from jax.experimental import pallas as pl
from jax.experimental.pallas import tpu as pltpu

N_DEV = 32


def kernel(x, w_mat, scale_x, scale_w):
    m, k_loc = x.shape
    _, n = w_mat.shape
    m_chunk = m // N_DEV

    def body(x_ref, w_ref, sx_ref, sw_ref, out_ref,
             comm_ref, send_sems, recv_sems, credit_sem):
        d = lax.axis_index("i")
        left = lax.rem(d + N_DEV - 1, N_DEV)
        right = lax.rem(d + 1, N_DEV)

        barrier_sem = pltpu.get_barrier_semaphore()
        for nbr in (left, right):
            pl.semaphore_signal(barrier_sem, inc=1, device_id=(nbr,),
                                device_id_type=pl.DeviceIdType.MESH)
        pl.semaphore_wait(barrier_sem, 2)

        def partial_chunk(c):
            a = x_ref[pl.ds(c * m_chunk, m_chunk), :]
            return lax.dot_general(
                a, w_ref[:, :], (((1,), (0,)), ((), ())),
                preferred_element_type=jnp.int32)

        comm_ref[0] = partial_chunk(lax.rem(d + N_DEV - 1, N_DEV))

        for s in range(N_DEV - 1):
            send_slot = s % 2
            recv_slot = (s + 1) % 2
            if s >= 1:
                pl.semaphore_wait(credit_sem, 1)
            rdma = pltpu.make_async_remote_copy(
                src_ref=comm_ref.at[send_slot],
                dst_ref=comm_ref.at[recv_slot],
                send_sem=send_sems.at[send_slot],
                recv_sem=recv_sems.at[recv_slot],
                device_id=(right,),
                device_id_type=pl.DeviceIdType.MESH,
            )
            rdma.start()
            p = partial_chunk(lax.rem(d + 2 * N_DEV - s - 2, N_DEV))
            rdma.wait_send()
            if s < N_DEV - 2:
                pl.semaphore_signal(credit_sem, inc=1, device_id=(left,),
                                    device_id_type=pl.DeviceIdType.MESH)
            rdma.wait_recv()
            acc = comm_ref[recv_slot] + p
            if s < N_DEV - 2:
                comm_ref[recv_slot] = acc
            else:
                scale = sx_ref[0] * sw_ref[0]
                out_ref[:, :] = jnp.maximum(acc.astype(jnp.float32) * scale,
                                            0.0)

    return pl.pallas_call(
        body,
        out_shape=jax.ShapeDtypeStruct((m_chunk, n), jnp.float32),
        in_specs=[
            pl.BlockSpec(memory_space=pltpu.VMEM),
            pl.BlockSpec(memory_space=pltpu.VMEM),
            pl.BlockSpec(memory_space=pltpu.SMEM),
            pl.BlockSpec(memory_space=pltpu.SMEM),
        ],
        out_specs=pl.BlockSpec(memory_space=pltpu.VMEM),
        scratch_shapes=[
            pltpu.VMEM((2, m_chunk, n), jnp.int32),
            pltpu.SemaphoreType.DMA((2,)),
            pltpu.SemaphoreType.DMA((2,)),
            pltpu.SemaphoreType.REGULAR,
        ],
        compiler_params=pltpu.CompilerParams(collective_id=0),
    )(x, w_mat, scale_x, scale_w)


# device time: 1462010 ns/iter; 1.0463x vs baseline; 1.0463x over previous
import jax
import jax.numpy as jnp
from jax import lax
from jax.experimental import pallas as pl
from jax.experimental.pallas import tpu as pltpu

N_DEV = 32


def kernel(x, w_mat, scale_x, scale_w):
    m, k_loc = x.shape
    _, n = w_mat.shape
    m_chunk = m // N_DEV
    n_half = n // 2

    def body(x_ref, w_ref, sx_ref, sw_ref, out_ref,
             comm_r_ref, comm_l_ref,
             send_sems_r, recv_sems_r, send_sems_l, recv_sems_l,
             credit_r, credit_l):
        d = lax.axis_index("i")
        left = lax.rem(d + N_DEV - 1, N_DEV)
        right = lax.rem(d + 1, N_DEV)

        barrier_sem = pltpu.get_barrier_semaphore()
        for nbr in (left, right):
            pl.semaphore_signal(barrier_sem, inc=1, device_id=(nbr,),
                                device_id_type=pl.DeviceIdType.MESH)
        pl.semaphore_wait(barrier_sem, 2)

        def partial_r(c):
            a = x_ref[pl.ds(c * m_chunk, m_chunk), :]
            return lax.dot_general(
                a, w_ref[:, 0:n_half], (((1,), (0,)), ((), ())),
                preferred_element_type=jnp.int32)

        def partial_l(c):
            a = x_ref[pl.ds(c * m_chunk, m_chunk), :]
            return lax.dot_general(
                a, w_ref[:, n_half:n], (((1,), (0,)), ((), ())),
                preferred_element_type=jnp.int32)

        comm_r_ref[0] = partial_r(lax.rem(d + N_DEV - 1, N_DEV))
        comm_l_ref[0] = partial_l(lax.rem(d + 1, N_DEV))

        for s in range(N_DEV - 1):
            send_slot = s % 2
            recv_slot = (s + 1) % 2
            if s >= 1:
                pl.semaphore_wait(credit_r, 1)
                pl.semaphore_wait(credit_l, 1)
            rdma_r = pltpu.make_async_remote_copy(
                src_ref=comm_r_ref.at[send_slot],
                dst_ref=comm_r_ref.at[recv_slot],
                send_sem=send_sems_r.at[send_slot],
                recv_sem=recv_sems_r.at[recv_slot],
                device_id=(right,),
                device_id_type=pl.DeviceIdType.MESH,
            )
            rdma_l = pltpu.make_async_remote_copy(
                src_ref=comm_l_ref.at[send_slot],
                dst_ref=comm_l_ref.at[recv_slot],
                send_sem=send_sems_l.at[send_slot],
                recv_sem=recv_sems_l.at[recv_slot],
                device_id=(left,),
                device_id_type=pl.DeviceIdType.MESH,
            )
            rdma_r.start()
            rdma_l.start()
            p_r = partial_r(lax.rem(d + 2 * N_DEV - s - 2, N_DEV))
            p_l = partial_l(lax.rem(d + s + 2, N_DEV))
            rdma_r.wait_send()
            rdma_l.wait_send()
            if s < N_DEV - 2:
                pl.semaphore_signal(credit_r, inc=1, device_id=(left,),
                                    device_id_type=pl.DeviceIdType.MESH)
                pl.semaphore_signal(credit_l, inc=1, device_id=(right,),
                                    device_id_type=pl.DeviceIdType.MESH)
            rdma_r.wait_recv()
            rdma_l.wait_recv()
            acc_r = comm_r_ref[recv_slot] + p_r
            acc_l = comm_l_ref[recv_slot] + p_l
            if s < N_DEV - 2:
                comm_r_ref[recv_slot] = acc_r
                comm_l_ref[recv_slot] = acc_l
            else:
                scale = sx_ref[0] * sw_ref[0]
                out_ref[:, 0:n_half] = jnp.maximum(
                    acc_r.astype(jnp.float32) * scale, 0.0)
                out_ref[:, n_half:n] = jnp.maximum(
                    acc_l.astype(jnp.float32) * scale, 0.0)

    return pl.pallas_call(
        body,
        out_shape=jax.ShapeDtypeStruct((m_chunk, n), jnp.float32),
        in_specs=[
            pl.BlockSpec(memory_space=pltpu.VMEM),
            pl.BlockSpec(memory_space=pltpu.VMEM),
            pl.BlockSpec(memory_space=pltpu.SMEM),
            pl.BlockSpec(memory_space=pltpu.SMEM),
        ],
        out_specs=pl.BlockSpec(memory_space=pltpu.VMEM),
        scratch_shapes=[
            pltpu.VMEM((2, m_chunk, n_half), jnp.int32),
            pltpu.VMEM((2, m_chunk, n_half), jnp.int32),
            pltpu.SemaphoreType.DMA((2,)),
            pltpu.SemaphoreType.DMA((2,)),
            pltpu.SemaphoreType.DMA((2,)),
            pltpu.SemaphoreType.DMA((2,)),
            pltpu.SemaphoreType.REGULAR,
            pltpu.SemaphoreType.REGULAR,
        ],
        compiler_params=pltpu.CompilerParams(collective_id=0),
    )(x, w_mat, scale_x, scale_w)


# device time: 779438 ns/iter; 1.9625x vs baseline; 1.8757x over previous
import jax
import jax.numpy as jnp
from jax import lax
from jax.experimental import pallas as pl
from jax.experimental.pallas import tpu as pltpu

N_DEV = 32

PERM = [0, 1, 9, 8, 16, 17, 25, 24, 27, 26, 18, 19, 11, 10, 13, 12,
        20, 21, 29, 28, 31, 30, 22, 23, 15, 14, 6, 7, 4, 5, 2, 3]
assert sorted(PERM) == list(range(N_DEV))
INV = [0] * N_DEV
for _j, _p in enumerate(PERM):
    INV[_p] = _j


def kernel(x, w_mat, scale_x, scale_w):
    m, k_loc = x.shape
    _, n = w_mat.shape
    m_chunk = m // N_DEV
    n_half = n // 2

    perm_arr = jnp.array(PERM, dtype=jnp.int32)
    inv_arr = jnp.array(INV, dtype=jnp.int32)
    d = lax.axis_index("i")
    j = inv_arr[d]
    s_arr = jnp.arange(N_DEV - 1, dtype=jnp.int32)
    left_right = jnp.stack([perm_arr[(j - 1) % N_DEV],
                            perm_arr[(j + 1) % N_DEV]])
    sched_r = perm_arr[(j - 2 - s_arr) % N_DEV]
    sched_l = perm_arr[(j + 2 + s_arr) % N_DEV]

    def body(x_ref, w_ref, sx_ref, sw_ref, lr_ref, schr_ref, schl_ref,
             out_ref, comm_r_ref, comm_l_ref,
             send_sems_r, recv_sems_r, send_sems_l, recv_sems_l,
             credit_r, credit_l):
        left = lr_ref[0]
        right = lr_ref[1]

        barrier_sem = pltpu.get_barrier_semaphore()
        for nbr in (left, right):
            pl.semaphore_signal(barrier_sem, inc=1, device_id=(nbr,),
                                device_id_type=pl.DeviceIdType.MESH)
        pl.semaphore_wait(barrier_sem, 2)

        def partial_r(c):
            a = x_ref[pl.ds(c * m_chunk, m_chunk), :]
            return lax.dot_general(
                a, w_ref[:, 0:n_half], (((1,), (0,)), ((), ())),
                preferred_element_type=jnp.int32)

        def partial_l(c):
            a = x_ref[pl.ds(c * m_chunk, m_chunk), :]
            return lax.dot_general(
                a, w_ref[:, n_half:n], (((1,), (0,)), ((), ())),
                preferred_element_type=jnp.int32)

        comm_r_ref[0] = partial_r(lr_ref[0])
        comm_l_ref[0] = partial_l(lr_ref[1])

        for s in range(N_DEV - 1):
            send_slot = s % 2
            recv_slot = (s + 1) % 2
            if s >= 1:
                pl.semaphore_wait(credit_r, 1)
                pl.semaphore_wait(credit_l, 1)
            rdma_r = pltpu.make_async_remote_copy(
                src_ref=comm_r_ref.at[send_slot],
                dst_ref=comm_r_ref.at[recv_slot],
                send_sem=send_sems_r.at[send_slot],
                recv_sem=recv_sems_r.at[recv_slot],
                device_id=(right,),
                device_id_type=pl.DeviceIdType.MESH,
            )
            rdma_l = pltpu.make_async_remote_copy(
                src_ref=comm_l_ref.at[send_slot],
                dst_ref=comm_l_ref.at[recv_slot],
                send_sem=send_sems_l.at[send_slot],
                recv_sem=recv_sems_l.at[recv_slot],
                device_id=(left,),
                device_id_type=pl.DeviceIdType.MESH,
            )
            rdma_r.start()
            rdma_l.start()
            p_r = partial_r(schr_ref[s])
            p_l = partial_l(schl_ref[s])
            rdma_r.wait_send()
            rdma_l.wait_send()
            if s < N_DEV - 2:
                pl.semaphore_signal(credit_r, inc=1, device_id=(left,),
                                    device_id_type=pl.DeviceIdType.MESH)
                pl.semaphore_signal(credit_l, inc=1, device_id=(right,),
                                    device_id_type=pl.DeviceIdType.MESH)
            rdma_r.wait_recv()
            rdma_l.wait_recv()
            acc_r = comm_r_ref[recv_slot] + p_r
            acc_l = comm_l_ref[recv_slot] + p_l
            if s < N_DEV - 2:
                comm_r_ref[recv_slot] = acc_r
                comm_l_ref[recv_slot] = acc_l
            else:
                scale = sx_ref[0] * sw_ref[0]
                out_ref[:, 0:n_half] = jnp.maximum(
                    acc_r.astype(jnp.float32) * scale, 0.0)
                out_ref[:, n_half:n] = jnp.maximum(
                    acc_l.astype(jnp.float32) * scale, 0.0)

    return pl.pallas_call(
        body,
        out_shape=jax.ShapeDtypeStruct((m_chunk, n), jnp.float32),
        in_specs=[
            pl.BlockSpec(memory_space=pltpu.VMEM),
            pl.BlockSpec(memory_space=pltpu.VMEM),
            pl.BlockSpec(memory_space=pltpu.SMEM),
            pl.BlockSpec(memory_space=pltpu.SMEM),
            pl.BlockSpec(memory_space=pltpu.SMEM),
            pl.BlockSpec(memory_space=pltpu.SMEM),
            pl.BlockSpec(memory_space=pltpu.SMEM),
        ],
        out_specs=pl.BlockSpec(memory_space=pltpu.VMEM),
        scratch_shapes=[
            pltpu.VMEM((2, m_chunk, n_half), jnp.int32),
            pltpu.VMEM((2, m_chunk, n_half), jnp.int32),
            pltpu.SemaphoreType.DMA((2,)),
            pltpu.SemaphoreType.DMA((2,)),
            pltpu.SemaphoreType.DMA((2,)),
            pltpu.SemaphoreType.DMA((2,)),
            pltpu.SemaphoreType.REGULAR,
            pltpu.SemaphoreType.REGULAR,
        ],
        compiler_params=pltpu.CompilerParams(collective_id=0),
    )(x, w_mat, scale_x, scale_w, left_right, sched_r, sched_l)


# device time: 717394 ns/iter; 2.1323x vs baseline; 1.0865x over previous
import jax
import jax.numpy as jnp
from jax import lax
from jax.experimental import pallas as pl
from jax.experimental.pallas import tpu as pltpu

N_DEV = 32
N_SUB = 4

PERM = [0, 1, 9, 8, 16, 17, 25, 24, 27, 26, 18, 19, 11, 10, 13, 12,
        20, 21, 29, 28, 31, 30, 22, 23, 15, 14, 6, 7, 4, 5, 2, 3]
assert sorted(PERM) == list(range(N_DEV))
INV = [0] * N_DEV
for _j, _p in enumerate(PERM):
    INV[_p] = _j


def kernel(x, w_mat, scale_x, scale_w):
    m, k_loc = x.shape
    _, n = w_mat.shape
    m_chunk = m // N_DEV
    n_sub = n // N_SUB

    perm_arr = jnp.array(PERM, dtype=jnp.int32)
    inv_arr = jnp.array(INV, dtype=jnp.int32)
    d = lax.axis_index("i")
    j = inv_arr[d]
    s_arr = jnp.arange(N_DEV - 1, dtype=jnp.int32)
    left_right = jnp.stack([perm_arr[(j - 1) % N_DEV],
                            perm_arr[(j + 1) % N_DEV]])
    sched_r = jnp.concatenate([perm_arr[(j - 2 - s_arr) % N_DEV],
                               perm_arr[(j - 1) % N_DEV][None]])
    sched_l = jnp.concatenate([perm_arr[(j + 2 + s_arr) % N_DEV],
                               perm_arr[(j + 1) % N_DEV][None]])

    def body(x_ref, w_ref, sx_ref, sw_ref, lr_ref, schr_ref, schl_ref,
             out_ref, comm_ref, send_sems, recv_sems, credit_sems):
        left = lr_ref[0]
        right = lr_ref[1]

        sub_dst = (right, right, left, left)
        sub_src = (left, left, right, right)
        sub_sched = (schr_ref, schr_ref, schl_ref, schl_ref)

        barrier_sem = pltpu.get_barrier_semaphore()
        for nbr in (left, right):
            pl.semaphore_signal(barrier_sem, inc=1, device_id=(nbr,),
                                device_id_type=pl.DeviceIdType.MESH)
        pl.semaphore_wait(barrier_sem, 2)

        def partial(u, c):
            a = x_ref[pl.ds(c * m_chunk, m_chunk), :]
            return lax.dot_general(
                a, w_ref[:, u * n_sub:(u + 1) * n_sub],
                (((1,), (0,)), ((), ())),
                preferred_element_type=jnp.int32)

        def make_rdma(u, s):
            return pltpu.make_async_remote_copy(
                src_ref=comm_ref.at[u, s % 2],
                dst_ref=comm_ref.at[u, (s + 1) % 2],
                send_sem=send_sems.at[u, s % 2],
                recv_sem=recv_sems.at[u, (s + 1) % 2],
                device_id=(sub_dst[u],),
                device_id_type=pl.DeviceIdType.MESH,
            )

        rdmas = [None] * N_SUB
        for u in range(N_SUB):
            comm_ref[u, 0] = partial(u, sub_sched[u][N_DEV - 1])
            rdmas[u] = make_rdma(u, 0)
            rdmas[u].start()

        for s in range(N_DEV - 1):
            recv_slot = (s + 1) % 2
            for u in range(N_SUB):
                p = partial(u, sub_sched[u][s])
                rdmas[u].wait_send()
                if s < N_DEV - 2:
                    pl.semaphore_signal(
                        credit_sems.at[u], inc=1, device_id=(sub_src[u],),
                        device_id_type=pl.DeviceIdType.MESH)
                rdmas[u].wait_recv()
                acc = comm_ref[u, recv_slot] + p
                if s < N_DEV - 2:
                    comm_ref[u, recv_slot] = acc
                    pl.semaphore_wait(credit_sems.at[u], 1)
                    rdmas[u] = make_rdma(u, s + 1)
                    rdmas[u].start()
                else:
                    scale = sx_ref[0] * sw_ref[0]
                    out_ref[:, u * n_sub:(u + 1) * n_sub] = jnp.maximum(
                        acc.astype(jnp.float32) * scale, 0.0)

    return pl.pallas_call(
        body,
        out_shape=jax.ShapeDtypeStruct((m_chunk, n), jnp.float32),
        in_specs=[
            pl.BlockSpec(memory_space=pltpu.VMEM),
            pl.BlockSpec(memory_space=pltpu.VMEM),
            pl.BlockSpec(memory_space=pltpu.SMEM),
            pl.BlockSpec(memory_space=pltpu.SMEM),
            pl.BlockSpec(memory_space=pltpu.SMEM),
            pl.BlockSpec(memory_space=pltpu.SMEM),
            pl.BlockSpec(memory_space=pltpu.SMEM),
        ],
        out_specs=pl.BlockSpec(memory_space=pltpu.VMEM),
        scratch_shapes=[
            pltpu.VMEM((N_SUB, 2, m_chunk, n_sub), jnp.int32),
            pltpu.SemaphoreType.DMA((N_SUB, 2)),
            pltpu.SemaphoreType.DMA((N_SUB, 2)),
            pltpu.SemaphoreType.REGULAR((N_SUB,)),
        ],
        compiler_params=pltpu.CompilerParams(collective_id=0),
    )(x, w_mat, scale_x, scale_w, left_right, sched_r, sched_l)
